# baseline (speedup 1.0000x reference)
import functools

import jax
import jax.numpy as jnp
from jax import lax
from jax.experimental import pallas as pl
from jax.experimental.pallas import tpu as pltpu
from jax.experimental.pallas import tpu_sc as plsc

N = 10000
NP = 10240
E = 320000
F_IN = 128
F96 = 96
F_H = 32
F_OUT = 45

CHUNK = 128
TILES = 32
G_B1 = 80
EPAD = TILES * G_B1 * CHUNK
ROWS_ALL = EPAD // CHUNK
G_B0 = ROWS_ALL // 16
NSLICE = NP // 16


def _zeros16():
    return jnp.zeros((16,), jnp.float32)


def _newton_rsqrt(d):
    i = plsc.bitcast(d, jnp.int32)
    i = jnp.int32(0x5F3759DF) - (i >> 1)
    y = plsc.bitcast(i, jnp.float32)
    for _ in range(3):
        y = y * (1.5 - 0.5 * d * y * y)
    return y


def _b0_body(dst_ref, w_ref, deg_ref, deg_acc, dv, wv, zbuf):
    c = lax.axis_index("c")
    s = lax.axis_index("s")
    wid = s * 2 + c

    def zb(i, _):
        zbuf[pl.ds(i * 16, 16)] = _zeros16()
        return _
    lax.fori_loop(0, NSLICE // 16, zb, None)
    pltpu.sync_copy(zbuf, deg_acc.at[pl.ds(s * NSLICE, NSLICE)])
    pltpu.sync_copy(dst_ref.at[pl.ds(wid * G_B1, G_B1)], dv)
    pltpu.sync_copy(w_ref.at[pl.ds(wid * G_B1, G_B1)], wv)
    plsc.subcore_barrier()

    def scat(j, _):
        pltpu.sync_copy(wv.at[j], deg_acc.at[dv.at[j]], add=True)
        return _
    lax.fori_loop(0, G_B1, scat, None)
    plsc.subcore_barrier()
    pltpu.sync_copy(deg_acc.at[pl.ds(s * NSLICE, NSLICE)],
                    deg_ref.at[c, pl.ds(s * NSLICE, NSLICE)])


_SC_PARAMS = pltpu.CompilerParams(needs_layout_passes=False,
                                  use_tc_tiling_on_sc=False)

_b0_kernel = functools.partial(
    pl.kernel,
    mesh=plsc.VectorSubcoreMesh(core_axis_name="c", subcore_axis_name="s"),
    compiler_params=_SC_PARAMS,
    out_type=jax.ShapeDtypeStruct((2, NP), jnp.float32),
    scratch_types=[
        pltpu.VMEM_SHARED((NP,), jnp.float32),
        pltpu.VMEM((G_B1, CHUNK), jnp.int32),
        pltpu.VMEM((G_B1, CHUNK), jnp.float32),
        pltpu.VMEM((NSLICE,), jnp.float32),
    ],
)(_b0_body)


FH2 = F96 // 2


STG = 80


def _b1_body(src_ref, dst_ref, w_ref, hall0_ref, hall1_ref, degp_ref, acc_ref,
             hd_half, acc_half, srcv, dstv, wv, rows0, rows1, zbuf,
             dgbuf, dinv_t, stbuf, gsem0, gsem1, ssem0, ssem1):
    c = lax.axis_index("c")
    s = lax.axis_index("s")
    wid = s * 2 + c
    row0 = wid * G_B1

    pltpu.sync_copy(src_ref.at[pl.ds(row0, G_B1)], srcv)
    pltpu.sync_copy(dst_ref.at[pl.ds(row0, G_B1)], dstv)
    pltpu.sync_copy(w_ref.at[pl.ds(row0, G_B1)], wv)

    pltpu.sync_copy(degp_ref.at[0, pl.ds(s * NSLICE, NSLICE)], dgbuf)
    pltpu.sync_copy(degp_ref.at[1, pl.ds(s * NSLICE, NSLICE)], dinv_t)

    def dvi(i, _):
        d = dgbuf[pl.ds(i * 16, 16)] + dinv_t[pl.ds(i * 16, 16)] + 1.0
        dinv_t[pl.ds(i * 16, 16)] = _newton_rsqrt(d)
        return _
    lax.fori_loop(0, NSLICE // 16, dvi, None)

    def zb(i, _):
        for j in range(FH2 // 16):
            zbuf[i, pl.ds(j * 16, 16)] = _zeros16()
        return _
    lax.fori_loop(0, 16, zb, None)

    def scale_buf(g, rows):
        def sc(e4, _):
            for de in range(4):
                e = e4 * 4 + de
                sp = plsc.load_gather(
                    wv, [jnp.full((16,), g, jnp.int32),
                         jnp.full((16,), e, jnp.int32)])
                for j in range(FH2 // 16):
                    rows[e, pl.ds(j * 16, 16)] = rows[e, pl.ds(j * 16, 16)] * sp
            return _
        lax.fori_loop(0, CHUNK // 4, sc, None)

    for h, hall_h in ((0, hall0_ref), (1, hall1_ref)):
        def zc(i, _):
            pltpu.sync_copy(zbuf, acc_half.at[pl.ds(s * NSLICE + i * 16, 16)])
            return _
        lax.fori_loop(0, NSLICE // 16, zc, None)

        def stg(i, _):
            base = s * NSLICE + i * STG
            pltpu.sync_copy(hall_h.at[pl.ds(base, STG)], stbuf)
            for r in range(STG):
                sp = plsc.load_gather(
                    dinv_t, [jnp.full((16,), i * STG + r, jnp.int32)])
                for j in range(FH2 // 16):
                    stbuf[r, pl.ds(j * 16, 16)] = stbuf[r, pl.ds(j * 16, 16)] * sp
            pltpu.sync_copy(stbuf, hd_half.at[pl.ds(base, STG)])
            return _
        nbatch = jnp.where(s == 15, (N - 15 * NSLICE) // STG, NSLICE // STG)
        lax.fori_loop(0, nbatch, stg, None)

        plsc.subcore_barrier()

        pltpu.async_copy(hd_half.at[srcv.at[0]], rows0, gsem0)
        pltpu.async_copy(hd_half.at[srcv.at[1]], rows1, gsem1)

        def pipe(gp, _):
            g0 = gp * 2
            g1 = g0 + 1
            pltpu.make_async_copy(hd_half.at[srcv.at[g0]], rows0, gsem0).wait()
            scale_buf(g0, rows0)
            pltpu.make_async_copy(hd_half.at[srcv.at[g1]], rows1, gsem1).wait()
            pltpu.async_copy(rows0, acc_half.at[dstv.at[g0]], ssem0, add=True)
            scale_buf(g1, rows1)
            pltpu.async_copy(rows1, acc_half.at[dstv.at[g1]], ssem1, add=True)

            @pl.when(g0 + 2 < G_B1)
            def _():
                pltpu.make_async_copy(rows0, acc_half.at[dstv.at[g0]], ssem0).wait()
                pltpu.async_copy(hd_half.at[srcv.at[g0 + 2]], rows0, gsem0)
                pltpu.make_async_copy(rows1, acc_half.at[dstv.at[g1]], ssem1).wait()
                pltpu.async_copy(hd_half.at[srcv.at[g1 + 2]], rows1, gsem1)
            return _
        lax.fori_loop(0, G_B1 // 2, pipe, None)
        pltpu.make_async_copy(rows0, acc_half.at[dstv.at[G_B1 - 2]], ssem0).wait()
        pltpu.make_async_copy(rows1, acc_half.at[dstv.at[G_B1 - 1]], ssem1).wait()
        plsc.subcore_barrier()

        pltpu.sync_copy(acc_half.at[pl.ds(s * NSLICE, NSLICE)],
                        acc_ref.at[c, h, pl.ds(s * NSLICE, NSLICE)])


_b1_kernel = functools.partial(
    pl.kernel,
    mesh=plsc.VectorSubcoreMesh(core_axis_name="c", subcore_axis_name="s"),
    compiler_params=_SC_PARAMS,
    out_type=jax.ShapeDtypeStruct((2, 2, NP, FH2), jnp.float32),
    scratch_types=[
        pltpu.VMEM_SHARED((NP, FH2), jnp.float32),
        pltpu.VMEM_SHARED((NP, FH2), jnp.float32),
        pltpu.VMEM((G_B1, CHUNK), jnp.int32),
        pltpu.VMEM((G_B1, CHUNK), jnp.int32),
        pltpu.VMEM((G_B1, CHUNK), jnp.float32),
        pltpu.VMEM((CHUNK, FH2), jnp.float32),
        pltpu.VMEM((CHUNK, FH2), jnp.float32),
        pltpu.VMEM((16, FH2), jnp.float32),
        pltpu.VMEM((NSLICE,), jnp.float32),
        pltpu.VMEM((NSLICE,), jnp.float32),
        pltpu.VMEM((STG, FH2), jnp.float32),
        pltpu.SemaphoreType.DMA,
        pltpu.SemaphoreType.DMA,
        pltpu.SemaphoreType.DMA,
        pltpu.SemaphoreType.DMA,
    ],
)(_b1_body)


def _a_body(x_ref, h_ref, wz_ref, wr_ref, wh_ref,
            lz_ref, lr_ref, lh_ref,
            hall_ref, hd0_ref, hd1_ref, hlz_ref, hlr_ref):
    lz1 = lz_ref[:, :F_H]
    lz2 = lz_ref[:, F_H:]
    lr1 = lr_ref[:, :F_H]
    lr2 = lr_ref[:, F_H:]
    lh1 = lh_ref[:, :F_H]
    wf = jnp.concatenate([
        jnp.dot(lz1, wz_ref[...], preferred_element_type=jnp.float32),
        jnp.dot(lr1, wr_ref[...], preferred_element_type=jnp.float32),
        jnp.dot(lh1, wh_ref[...], preferred_element_type=jnp.float32),
    ], axis=0)
    hall = jnp.dot(x_ref[...], wf.T, preferred_element_type=jnp.float32)
    hall_ref[...] = hall
    hd0_ref[...] = hall[:, :FH2]
    hd1_ref[...] = hall[:, FH2:]
    hlz_ref[...] = jnp.dot(h_ref[...], lz2.T, preferred_element_type=jnp.float32)
    hlr_ref[...] = jnp.dot(h_ref[...], lr2.T, preferred_element_type=jnp.float32)


def _c_body(a00_ref, a01_ref, a10_ref, a11_ref,
            hall_ref, hlz_ref, hlr_ref, dg0_ref, dg1_ref, h_ref,
            lz_ref, lr_ref, lh_ref, bz_ref, br_ref, bh_ref,
            lzb_ref, lrb_ref, lhb_ref, linw_ref, linb_ref, y_ref, hn_ref):
    dinv = lax.rsqrt(dg0_ref[...] + dg1_ref[...] + 1.0)
    hall = hall_ref[...]
    acc = jnp.concatenate([a00_ref[...] + a10_ref[...],
                           a01_ref[...] + a11_ref[...]], axis=1)
    agg = dinv * acc + dinv * dinv * hall
    lz1 = lz_ref[:, :F_H]
    lr1 = lr_ref[:, :F_H]
    lh1 = lh_ref[:, :F_H]
    lh2 = lh_ref[:, F_H:]
    bzf = jnp.dot(bz_ref[...], lz1.T, preferred_element_type=jnp.float32) + lzb_ref[...]
    brf = jnp.dot(br_ref[...], lr1.T, preferred_element_type=jnp.float32) + lrb_ref[...]
    bhf = jnp.dot(bh_ref[...], lh1.T, preferred_element_type=jnp.float32) + lhb_ref[...]
    h = h_ref[...]
    z = jax.nn.sigmoid(agg[:, :F_H] + hlz_ref[...] + bzf)
    r = jax.nn.sigmoid(agg[:, F_H:2 * F_H] + hlr_ref[...] + brf)
    ht = jnp.tanh(agg[:, 2 * F_H:] + jnp.dot(h * r, lh2.T, preferred_element_type=jnp.float32) + bhf)
    hn = z * h + (1.0 - z) * ht
    hn_ref[...] = hn
    y_ref[...] = (jnp.dot(jnp.maximum(hn, 0.0), linw_ref[...].T,
                          preferred_element_type=jnp.float32) + linb_ref[...])


def kernel(x, edge_index, edge_weight, prev_hidden_state,
           Wz, bz, Lz_W, Lz_b, Wr, br, Lr_W, Lr_b,
           Wh, bh, Lh_W, Lh_b, lin_W, lin_b):
    src = edge_index[0]
    dst = edge_index[1]
    pad = EPAD - E
    src2 = jnp.concatenate([src, jnp.zeros((pad,), src.dtype)]).reshape(ROWS_ALL, CHUNK)
    dst2 = jnp.concatenate([dst, jnp.zeros((pad,), dst.dtype)]).reshape(ROWS_ALL, CHUNK)
    w2 = jnp.concatenate([edge_weight, jnp.zeros((pad,), edge_weight.dtype)]).reshape(ROWS_ALL, CHUNK)

    bn = 1000
    grid = (N // bn,)
    full = lambda shp: pl.BlockSpec(shp, lambda i: (0, 0))
    hall, hd0, hd1, hlz, hlr = pl.pallas_call(
        _a_body,
        grid=grid,
        in_specs=[
            pl.BlockSpec((bn, F_IN), lambda i: (i, 0)),
            pl.BlockSpec((bn, F_H), lambda i: (i, 0)),
            full((F_H, F_IN)), full((F_H, F_IN)), full((F_H, F_IN)),
            full((F_H, 2 * F_H)), full((F_H, 2 * F_H)), full((F_H, 2 * F_H)),
        ],
        out_specs=[
            pl.BlockSpec((bn, F96), lambda i: (i, 0)),
            pl.BlockSpec((bn, FH2), lambda i: (i, 0)),
            pl.BlockSpec((bn, FH2), lambda i: (i, 0)),
            pl.BlockSpec((bn, F_H), lambda i: (i, 0)),
            pl.BlockSpec((bn, F_H), lambda i: (i, 0)),
        ],
        out_shape=[
            jax.ShapeDtypeStruct((N, F96), jnp.float32),
            jax.ShapeDtypeStruct((N, FH2), jnp.float32),
            jax.ShapeDtypeStruct((N, FH2), jnp.float32),
            jax.ShapeDtypeStruct((N, F_H), jnp.float32),
            jax.ShapeDtypeStruct((N, F_H), jnp.float32),
        ],
    )(x, prev_hidden_state, Wz, Wr, Wh, Lz_W, Lr_W, Lh_W)

    deg_p = _b0_kernel(dst2, w2)
    dg0 = deg_p[0, :N].reshape(N, 1)
    dg1 = deg_p[1, :N].reshape(N, 1)

    accp = _b1_kernel(src2, dst2, w2, hd0, hd1, deg_p)

    a00 = accp[0, 0]
    a01 = accp[0, 1]
    a10 = accp[1, 0]
    a11 = accp[1, 1]
    bz1 = bz.reshape(1, F_H)
    br1 = br.reshape(1, F_H)
    bh1 = bh.reshape(1, F_H)
    lzb1 = Lz_b.reshape(1, F_H)
    lrb1 = Lr_b.reshape(1, F_H)
    lhb1 = Lh_b.reshape(1, F_H)
    linb1 = lin_b.reshape(1, F_OUT)

    y, hn = pl.pallas_call(
        _c_body,
        grid=grid,
        in_specs=[
            pl.BlockSpec((bn, FH2), lambda i: (i, 0)),
            pl.BlockSpec((bn, FH2), lambda i: (i, 0)),
            pl.BlockSpec((bn, FH2), lambda i: (i, 0)),
            pl.BlockSpec((bn, FH2), lambda i: (i, 0)),
            pl.BlockSpec((bn, F96), lambda i: (i, 0)),
            pl.BlockSpec((bn, F_H), lambda i: (i, 0)),
            pl.BlockSpec((bn, F_H), lambda i: (i, 0)),
            pl.BlockSpec((bn, 1), lambda i: (i, 0)),
            pl.BlockSpec((bn, 1), lambda i: (i, 0)),
            pl.BlockSpec((bn, F_H), lambda i: (i, 0)),
            full((F_H, 2 * F_H)),
            full((F_H, 2 * F_H)),
            full((F_H, 2 * F_H)),
            full((1, F_H)), full((1, F_H)), full((1, F_H)),
            full((1, F_H)), full((1, F_H)), full((1, F_H)),
            full((F_OUT, F_H)),
            full((1, F_OUT)),
        ],
        out_specs=[
            pl.BlockSpec((bn, F_OUT), lambda i: (i, 0)),
            pl.BlockSpec((bn, F_H), lambda i: (i, 0)),
        ],
        out_shape=[
            jax.ShapeDtypeStruct((N, F_OUT), jnp.float32),
            jax.ShapeDtypeStruct((N, F_H), jnp.float32),
        ],
    )(a00, a01, a10, a11, hall, hlz, hlr, dg0, dg1, prev_hidden_state,
      Lz_W, Lr_W, Lh_W, bz1, br1, bh1, lzb1, lrb1, lhb1,
      lin_W, linb1)

    return (y, hn)


# speedup vs baseline: 1.0140x; 1.0140x over previous
import functools

import jax
import jax.numpy as jnp
from jax import lax
from jax.experimental import pallas as pl
from jax.experimental.pallas import tpu as pltpu
from jax.experimental.pallas import tpu_sc as plsc

N = 10000
NP = 10240
E = 320000
F_IN = 128
F96 = 96
F_H = 32
F_OUT = 45

CHUNK = 128
TILES = 32
G_B1 = 80
EPAD = TILES * G_B1 * CHUNK
ROWS_ALL = EPAD // CHUNK
G_B0 = ROWS_ALL // 16
NSLICE = NP // 16


def _zeros16():
    return jnp.zeros((16,), jnp.float32)


def _b0_body(dst_ref, w_ref, deg_ref, deg_acc, dv, wv, zbuf):
    c = lax.axis_index("c")
    s = lax.axis_index("s")
    wid = s * 2 + c

    def zb(i, _):
        zbuf[pl.ds(i * 16, 16)] = _zeros16()
        return _
    lax.fori_loop(0, NSLICE // 16, zb, None)
    pltpu.sync_copy(zbuf, deg_acc.at[pl.ds(s * NSLICE, NSLICE)])
    pltpu.sync_copy(dst_ref.at[pl.ds(wid * G_B1, G_B1)], dv)
    pltpu.sync_copy(w_ref.at[pl.ds(wid * G_B1, G_B1)], wv)
    plsc.subcore_barrier()

    def scat(j, _):
        pltpu.sync_copy(wv.at[j], deg_acc.at[dv.at[j]], add=True)
        return _
    lax.fori_loop(0, G_B1, scat, None)
    plsc.subcore_barrier()
    pltpu.sync_copy(deg_acc.at[pl.ds(s * NSLICE, NSLICE)],
                    deg_ref.at[c, pl.ds(s * NSLICE, NSLICE)])


_SC_PARAMS = pltpu.CompilerParams(needs_layout_passes=False,
                                  use_tc_tiling_on_sc=False,
                                  skip_device_barrier=True)

_b0_kernel = functools.partial(
    pl.kernel,
    mesh=plsc.VectorSubcoreMesh(core_axis_name="c", subcore_axis_name="s"),
    compiler_params=_SC_PARAMS,
    out_type=jax.ShapeDtypeStruct((2, NP), jnp.float32),
    scratch_types=[
        pltpu.VMEM_SHARED((NP,), jnp.float32),
        pltpu.VMEM((G_B1, CHUNK), jnp.int32),
        pltpu.VMEM((G_B1, CHUNK), jnp.float32),
        pltpu.VMEM((NSLICE,), jnp.float32),
    ],
)(_b0_body)


FH2 = F96 // 2


def _b1_body(src_ref, dst_ref, w_ref, hd0_ref, hd1_ref, acc_ref,
             hd_half, acc_half, srcv, dstv, wv, rows0, rows1, zbuf,
             gsem0, gsem1, ssem0, ssem1):
    c = lax.axis_index("c")
    s = lax.axis_index("s")
    wid = s * 2 + c
    row0 = wid * G_B1

    pltpu.sync_copy(src_ref.at[pl.ds(row0, G_B1)], srcv)
    pltpu.sync_copy(dst_ref.at[pl.ds(row0, G_B1)], dstv)
    pltpu.sync_copy(w_ref.at[pl.ds(row0, G_B1)], wv)

    def zb(i, _):
        for j in range(FH2 // 16):
            zbuf[i, pl.ds(j * 16, 16)] = _zeros16()
        return _
    lax.fori_loop(0, 16, zb, None)

    def scale_buf(g, rows):
        def sc(e4, _):
            for de in range(4):
                e = e4 * 4 + de
                sp = plsc.load_gather(
                    wv, [jnp.full((16,), g, jnp.int32),
                         jnp.full((16,), e, jnp.int32)])
                for j in range(FH2 // 16):
                    rows[e, pl.ds(j * 16, 16)] = rows[e, pl.ds(j * 16, 16)] * sp
            return _
        lax.fori_loop(0, CHUNK // 4, sc, None)

    for h, hd_h in ((0, hd0_ref), (1, hd1_ref)):
        def zc(i, _):
            pltpu.sync_copy(zbuf, acc_half.at[pl.ds(s * NSLICE + i * 16, 16)])
            return _
        lax.fori_loop(0, NSLICE // 16, zc, None)

        @pl.when(s < 15)
        def _():
            pltpu.sync_copy(hd_h.at[pl.ds(s * NSLICE, NSLICE)],
                            hd_half.at[pl.ds(s * NSLICE, NSLICE)])

        @pl.when(s == 15)
        def _():
            pltpu.sync_copy(hd_h.at[pl.ds(15 * NSLICE, N - 15 * NSLICE)],
                            hd_half.at[pl.ds(15 * NSLICE, N - 15 * NSLICE)])

        plsc.subcore_barrier()

        pltpu.async_copy(hd_half.at[srcv.at[0]], rows0, gsem0)
        pltpu.async_copy(hd_half.at[srcv.at[1]], rows1, gsem1)

        def pipe(gp, _):
            g0 = gp * 2
            g1 = g0 + 1
            pltpu.make_async_copy(hd_half.at[srcv.at[g0]], rows0, gsem0).wait()
            scale_buf(g0, rows0)
            pltpu.make_async_copy(hd_half.at[srcv.at[g1]], rows1, gsem1).wait()
            pltpu.async_copy(rows0, acc_half.at[dstv.at[g0]], ssem0, add=True)
            scale_buf(g1, rows1)
            pltpu.async_copy(rows1, acc_half.at[dstv.at[g1]], ssem1, add=True)

            @pl.when(g0 + 2 < G_B1)
            def _():
                pltpu.make_async_copy(rows0, acc_half.at[dstv.at[g0]], ssem0).wait()
                pltpu.async_copy(hd_half.at[srcv.at[g0 + 2]], rows0, gsem0)
                pltpu.make_async_copy(rows1, acc_half.at[dstv.at[g1]], ssem1).wait()
                pltpu.async_copy(hd_half.at[srcv.at[g1 + 2]], rows1, gsem1)
            return _
        lax.fori_loop(0, G_B1 // 2, pipe, None)
        pltpu.make_async_copy(rows0, acc_half.at[dstv.at[G_B1 - 2]], ssem0).wait()
        pltpu.make_async_copy(rows1, acc_half.at[dstv.at[G_B1 - 1]], ssem1).wait()
        plsc.subcore_barrier()

        pltpu.sync_copy(acc_half.at[pl.ds(s * NSLICE, NSLICE)],
                        acc_ref.at[c, h, pl.ds(s * NSLICE, NSLICE)])


_b1_kernel = functools.partial(
    pl.kernel,
    mesh=plsc.VectorSubcoreMesh(core_axis_name="c", subcore_axis_name="s"),
    compiler_params=_SC_PARAMS,
    out_type=jax.ShapeDtypeStruct((2, 2, NP, FH2), jnp.float32),
    scratch_types=[
        pltpu.VMEM_SHARED((NP, FH2), jnp.float32),
        pltpu.VMEM_SHARED((NP, FH2), jnp.float32),
        pltpu.VMEM((G_B1, CHUNK), jnp.int32),
        pltpu.VMEM((G_B1, CHUNK), jnp.int32),
        pltpu.VMEM((G_B1, CHUNK), jnp.float32),
        pltpu.VMEM((CHUNK, FH2), jnp.float32),
        pltpu.VMEM((CHUNK, FH2), jnp.float32),
        pltpu.VMEM((16, FH2), jnp.float32),
        pltpu.SemaphoreType.DMA,
        pltpu.SemaphoreType.DMA,
        pltpu.SemaphoreType.DMA,
        pltpu.SemaphoreType.DMA,
    ],
)(_b1_body)


def _a_body(x_ref, h_ref, dg0_ref, dg1_ref, wz_ref, wr_ref, wh_ref,
            lz_ref, lr_ref, lh_ref,
            hall_ref, hd0_ref, hd1_ref, hlz_ref, hlr_ref, dinv_ref):
    lz1 = lz_ref[:, :F_H]
    lz2 = lz_ref[:, F_H:]
    lr1 = lr_ref[:, :F_H]
    lr2 = lr_ref[:, F_H:]
    lh1 = lh_ref[:, :F_H]
    wf = jnp.concatenate([
        jnp.dot(lz1, wz_ref[...], preferred_element_type=jnp.float32),
        jnp.dot(lr1, wr_ref[...], preferred_element_type=jnp.float32),
        jnp.dot(lh1, wh_ref[...], preferred_element_type=jnp.float32),
    ], axis=0)
    hall = jnp.dot(x_ref[...], wf.T, preferred_element_type=jnp.float32)
    hall_ref[...] = hall
    dinv = lax.rsqrt(dg0_ref[...] + dg1_ref[...] + 1.0)
    dinv_ref[...] = dinv
    hd = hall * dinv
    hd0_ref[...] = hd[:, :FH2]
    hd1_ref[...] = hd[:, FH2:]
    hlz_ref[...] = jnp.dot(h_ref[...], lz2.T, preferred_element_type=jnp.float32)
    hlr_ref[...] = jnp.dot(h_ref[...], lr2.T, preferred_element_type=jnp.float32)


def _c_body(a00_ref, a01_ref, a10_ref, a11_ref,
            hall_ref, hlz_ref, hlr_ref, dinv_ref, h_ref,
            lz_ref, lr_ref, lh_ref, bz_ref, br_ref, bh_ref,
            lzb_ref, lrb_ref, lhb_ref, linw_ref, linb_ref, y_ref, hn_ref):
    dinv = dinv_ref[...]
    hall = hall_ref[...]
    acc = jnp.concatenate([a00_ref[...] + a10_ref[...],
                           a01_ref[...] + a11_ref[...]], axis=1)
    agg = dinv * acc + dinv * dinv * hall
    lz1 = lz_ref[:, :F_H]
    lr1 = lr_ref[:, :F_H]
    lh1 = lh_ref[:, :F_H]
    lh2 = lh_ref[:, F_H:]
    bzf = jnp.dot(bz_ref[...], lz1.T, preferred_element_type=jnp.float32) + lzb_ref[...]
    brf = jnp.dot(br_ref[...], lr1.T, preferred_element_type=jnp.float32) + lrb_ref[...]
    bhf = jnp.dot(bh_ref[...], lh1.T, preferred_element_type=jnp.float32) + lhb_ref[...]
    h = h_ref[...]
    z = jax.nn.sigmoid(agg[:, :F_H] + hlz_ref[...] + bzf)
    r = jax.nn.sigmoid(agg[:, F_H:2 * F_H] + hlr_ref[...] + brf)
    ht = jnp.tanh(agg[:, 2 * F_H:] + jnp.dot(h * r, lh2.T, preferred_element_type=jnp.float32) + bhf)
    hn = z * h + (1.0 - z) * ht
    hn_ref[...] = hn
    y_ref[...] = (jnp.dot(jnp.maximum(hn, 0.0), linw_ref[...].T,
                          preferred_element_type=jnp.float32) + linb_ref[...])


def kernel(x, edge_index, edge_weight, prev_hidden_state,
           Wz, bz, Lz_W, Lz_b, Wr, br, Lr_W, Lr_b,
           Wh, bh, Lh_W, Lh_b, lin_W, lin_b):
    src = edge_index[0]
    dst = edge_index[1]
    pad = EPAD - E
    src2 = jnp.concatenate([src, jnp.zeros((pad,), src.dtype)]).reshape(ROWS_ALL, CHUNK)
    dst2 = jnp.concatenate([dst, jnp.zeros((pad,), dst.dtype)]).reshape(ROWS_ALL, CHUNK)
    w2 = jnp.concatenate([edge_weight, jnp.zeros((pad,), edge_weight.dtype)]).reshape(ROWS_ALL, CHUNK)

    deg_p = _b0_kernel(dst2, w2)

    bn = 1000
    grid = (N // bn,)
    full = lambda shp: pl.BlockSpec(shp, lambda i: (0, 0))
    dg0 = deg_p[0, :N].reshape(N, 1)
    dg1 = deg_p[1, :N].reshape(N, 1)
    hall, hd0, hd1, hlz, hlr, dinv1 = pl.pallas_call(
        _a_body,
        grid=grid,
        in_specs=[
            pl.BlockSpec((bn, F_IN), lambda i: (i, 0)),
            pl.BlockSpec((bn, F_H), lambda i: (i, 0)),
            pl.BlockSpec((bn, 1), lambda i: (i, 0)),
            pl.BlockSpec((bn, 1), lambda i: (i, 0)),
            full((F_H, F_IN)), full((F_H, F_IN)), full((F_H, F_IN)),
            full((F_H, 2 * F_H)), full((F_H, 2 * F_H)), full((F_H, 2 * F_H)),
        ],
        out_specs=[
            pl.BlockSpec((bn, F96), lambda i: (i, 0)),
            pl.BlockSpec((bn, FH2), lambda i: (i, 0)),
            pl.BlockSpec((bn, FH2), lambda i: (i, 0)),
            pl.BlockSpec((bn, F_H), lambda i: (i, 0)),
            pl.BlockSpec((bn, F_H), lambda i: (i, 0)),
            pl.BlockSpec((bn, 1), lambda i: (i, 0)),
        ],
        out_shape=[
            jax.ShapeDtypeStruct((N, F96), jnp.float32),
            jax.ShapeDtypeStruct((N, FH2), jnp.float32),
            jax.ShapeDtypeStruct((N, FH2), jnp.float32),
            jax.ShapeDtypeStruct((N, F_H), jnp.float32),
            jax.ShapeDtypeStruct((N, F_H), jnp.float32),
            jax.ShapeDtypeStruct((N, 1), jnp.float32),
        ],
    )(x, prev_hidden_state, dg0, dg1, Wz, Wr, Wh, Lz_W, Lr_W, Lh_W)

    accp = _b1_kernel(src2, dst2, w2, hd0, hd1)

    a00 = accp[0, 0]
    a01 = accp[0, 1]
    a10 = accp[1, 0]
    a11 = accp[1, 1]
    bz1 = bz.reshape(1, F_H)
    br1 = br.reshape(1, F_H)
    bh1 = bh.reshape(1, F_H)
    lzb1 = Lz_b.reshape(1, F_H)
    lrb1 = Lr_b.reshape(1, F_H)
    lhb1 = Lh_b.reshape(1, F_H)
    linb1 = lin_b.reshape(1, F_OUT)

    y, hn = pl.pallas_call(
        _c_body,
        grid=grid,
        in_specs=[
            pl.BlockSpec((bn, FH2), lambda i: (i, 0)),
            pl.BlockSpec((bn, FH2), lambda i: (i, 0)),
            pl.BlockSpec((bn, FH2), lambda i: (i, 0)),
            pl.BlockSpec((bn, FH2), lambda i: (i, 0)),
            pl.BlockSpec((bn, F96), lambda i: (i, 0)),
            pl.BlockSpec((bn, F_H), lambda i: (i, 0)),
            pl.BlockSpec((bn, F_H), lambda i: (i, 0)),
            pl.BlockSpec((bn, 1), lambda i: (i, 0)),
            pl.BlockSpec((bn, F_H), lambda i: (i, 0)),
            full((F_H, 2 * F_H)),
            full((F_H, 2 * F_H)),
            full((F_H, 2 * F_H)),
            full((1, F_H)), full((1, F_H)), full((1, F_H)),
            full((1, F_H)), full((1, F_H)), full((1, F_H)),
            full((F_OUT, F_H)),
            full((1, F_OUT)),
        ],
        out_specs=[
            pl.BlockSpec((bn, F_OUT), lambda i: (i, 0)),
            pl.BlockSpec((bn, F_H), lambda i: (i, 0)),
        ],
        out_shape=[
            jax.ShapeDtypeStruct((N, F_OUT), jnp.float32),
            jax.ShapeDtypeStruct((N, F_H), jnp.float32),
        ],
    )(a00, a01, a10, a11, hall, hlz, hlr, dinv1, prev_hidden_state,
      Lz_W, Lr_W, Lh_W, bz1, br1, bh1, lzb1, lrb1, lhb1,
      lin_W, linb1)

    return (y, hn)


# speedup vs baseline: 1.0196x; 1.0055x over previous
import functools

import jax
import jax.numpy as jnp
from jax import lax
from jax.experimental import pallas as pl
from jax.experimental.pallas import tpu as pltpu
from jax.experimental.pallas import tpu_sc as plsc

N = 10000
NP = 10240
E = 320000
F_IN = 128
F96 = 96
F_H = 32
F_OUT = 45

CHUNK = 128
TILES = 32
G_B1 = 80
EPAD = TILES * G_B1 * CHUNK
ROWS_ALL = EPAD // CHUNK
G_B0 = ROWS_ALL // 16
NSLICE = NP // 16


def _zeros16():
    return jnp.zeros((16,), jnp.float32)


def _b0_body(dst_ref, w_ref, deg_ref, deg_acc, dv, wv, zbuf):
    c = lax.axis_index("c")
    s = lax.axis_index("s")
    wid = s * 2 + c

    def zb(i, _):
        zbuf[pl.ds(i * 16, 16)] = _zeros16()
        return _
    lax.fori_loop(0, NSLICE // 16, zb, None)
    pltpu.sync_copy(zbuf, deg_acc.at[pl.ds(s * NSLICE, NSLICE)])
    pltpu.sync_copy(dst_ref.at[pl.ds(wid * G_B1, G_B1)], dv)
    pltpu.sync_copy(w_ref.at[pl.ds(wid * G_B1, G_B1)], wv)
    plsc.subcore_barrier()

    def scat(j, _):
        pltpu.sync_copy(wv.at[j], deg_acc.at[dv.at[j]], add=True)
        return _
    lax.fori_loop(0, G_B1, scat, None)
    plsc.subcore_barrier()
    pltpu.sync_copy(deg_acc.at[pl.ds(s * NSLICE, NSLICE)],
                    deg_ref.at[c, pl.ds(s * NSLICE, NSLICE)])


_SC_PARAMS = pltpu.CompilerParams(needs_layout_passes=False,
                                  use_tc_tiling_on_sc=False)

_b0_kernel = functools.partial(
    pl.kernel,
    mesh=plsc.VectorSubcoreMesh(core_axis_name="c", subcore_axis_name="s"),
    compiler_params=_SC_PARAMS,
    out_type=jax.ShapeDtypeStruct((2, NP), jnp.float32),
    scratch_types=[
        pltpu.VMEM_SHARED((NP,), jnp.float32),
        pltpu.VMEM((G_B1, CHUNK), jnp.int32),
        pltpu.VMEM((G_B1, CHUNK), jnp.float32),
        pltpu.VMEM((NSLICE,), jnp.float32),
    ],
)(_b0_body)


FH2 = F96 // 2


def _b1_body(src_ref, dst_ref, w_ref, hd0_ref, hd1_ref, acc_ref,
             hd_half, acc_half, srcv, dstv, wv, rows0, rows1, zbuf,
             gsem0, gsem1, ssem0, ssem1):
    c = lax.axis_index("c")
    s = lax.axis_index("s")
    wid = s * 2 + c
    row0 = wid * G_B1

    pltpu.sync_copy(src_ref.at[pl.ds(row0, G_B1)], srcv)
    pltpu.sync_copy(dst_ref.at[pl.ds(row0, G_B1)], dstv)
    pltpu.sync_copy(w_ref.at[pl.ds(row0, G_B1)], wv)

    def zb(i, _):
        for j in range(FH2 // 16):
            zbuf[i, pl.ds(j * 16, 16)] = _zeros16()
        return _
    lax.fori_loop(0, 64, zb, None)

    def scale_buf(g, rows):
        def sc(e8, _):
            for de in range(8):
                e = e8 * 8 + de
                sp = plsc.load_gather(
                    wv, [jnp.full((16,), g, jnp.int32),
                         jnp.full((16,), e, jnp.int32)])
                for j in range(FH2 // 16):
                    rows[e, pl.ds(j * 16, 16)] = rows[e, pl.ds(j * 16, 16)] * sp
            return _
        lax.fori_loop(0, CHUNK // 8, sc, None)

    for h, hd_h in ((0, hd0_ref), (1, hd1_ref)):
        def zc(i, _):
            pltpu.sync_copy(zbuf, acc_half.at[pl.ds(s * NSLICE + i * 64, 64)])
            return _
        lax.fori_loop(0, NSLICE // 64, zc, None)

        @pl.when(s < 15)
        def _():
            pltpu.sync_copy(hd_h.at[pl.ds(s * NSLICE, NSLICE)],
                            hd_half.at[pl.ds(s * NSLICE, NSLICE)])

        @pl.when(s == 15)
        def _():
            pltpu.sync_copy(hd_h.at[pl.ds(15 * NSLICE, N - 15 * NSLICE)],
                            hd_half.at[pl.ds(15 * NSLICE, N - 15 * NSLICE)])

        plsc.subcore_barrier()

        pltpu.async_copy(hd_half.at[srcv.at[0]], rows0, gsem0)
        pltpu.async_copy(hd_half.at[srcv.at[1]], rows1, gsem1)

        def pipe(gp, _):
            g0 = gp * 2
            g1 = g0 + 1
            pltpu.make_async_copy(hd_half.at[srcv.at[g0]], rows0, gsem0).wait()
            scale_buf(g0, rows0)
            pltpu.make_async_copy(hd_half.at[srcv.at[g1]], rows1, gsem1).wait()
            pltpu.async_copy(rows0, acc_half.at[dstv.at[g0]], ssem0, add=True)
            scale_buf(g1, rows1)
            pltpu.async_copy(rows1, acc_half.at[dstv.at[g1]], ssem1, add=True)

            @pl.when(g0 + 2 < G_B1)
            def _():
                pltpu.make_async_copy(rows0, acc_half.at[dstv.at[g0]], ssem0).wait()
                pltpu.async_copy(hd_half.at[srcv.at[g0 + 2]], rows0, gsem0)
                pltpu.make_async_copy(rows1, acc_half.at[dstv.at[g1]], ssem1).wait()
                pltpu.async_copy(hd_half.at[srcv.at[g1 + 2]], rows1, gsem1)
            return _
        lax.fori_loop(0, G_B1 // 2, pipe, None)
        pltpu.make_async_copy(rows0, acc_half.at[dstv.at[G_B1 - 2]], ssem0).wait()
        pltpu.make_async_copy(rows1, acc_half.at[dstv.at[G_B1 - 1]], ssem1).wait()
        plsc.subcore_barrier()

        pltpu.sync_copy(acc_half.at[pl.ds(s * NSLICE, NSLICE)],
                        acc_ref.at[c, h, pl.ds(s * NSLICE, NSLICE)])


_b1_kernel = functools.partial(
    pl.kernel,
    mesh=plsc.VectorSubcoreMesh(core_axis_name="c", subcore_axis_name="s"),
    compiler_params=_SC_PARAMS,
    out_type=jax.ShapeDtypeStruct((2, 2, NP, FH2), jnp.float32),
    scratch_types=[
        pltpu.VMEM_SHARED((NP, FH2), jnp.float32),
        pltpu.VMEM_SHARED((NP, FH2), jnp.float32),
        pltpu.VMEM((G_B1, CHUNK), jnp.int32),
        pltpu.VMEM((G_B1, CHUNK), jnp.int32),
        pltpu.VMEM((G_B1, CHUNK), jnp.float32),
        pltpu.VMEM((CHUNK, FH2), jnp.float32),
        pltpu.VMEM((CHUNK, FH2), jnp.float32),
        pltpu.VMEM((64, FH2), jnp.float32),
        pltpu.SemaphoreType.DMA,
        pltpu.SemaphoreType.DMA,
        pltpu.SemaphoreType.DMA,
        pltpu.SemaphoreType.DMA,
    ],
)(_b1_body)


def _a_body(x_ref, h_ref, dg0_ref, dg1_ref, wz_ref, wr_ref, wh_ref,
            lz_ref, lr_ref, lh_ref,
            hall_ref, hd0_ref, hd1_ref, hlz_ref, hlr_ref, dinv_ref):
    lz1 = lz_ref[:, :F_H]
    lz2 = lz_ref[:, F_H:]
    lr1 = lr_ref[:, :F_H]
    lr2 = lr_ref[:, F_H:]
    lh1 = lh_ref[:, :F_H]
    wf = jnp.concatenate([
        jnp.dot(lz1, wz_ref[...], preferred_element_type=jnp.float32),
        jnp.dot(lr1, wr_ref[...], preferred_element_type=jnp.float32),
        jnp.dot(lh1, wh_ref[...], preferred_element_type=jnp.float32),
    ], axis=0)
    hall = jnp.dot(x_ref[...], wf.T, preferred_element_type=jnp.float32)
    hall_ref[...] = hall
    dinv = lax.rsqrt(dg0_ref[...] + dg1_ref[...] + 1.0)
    dinv_ref[...] = dinv
    hd = hall * dinv
    hd0_ref[...] = hd[:, :FH2]
    hd1_ref[...] = hd[:, FH2:]
    hlz_ref[...] = jnp.dot(h_ref[...], lz2.T, preferred_element_type=jnp.float32)
    hlr_ref[...] = jnp.dot(h_ref[...], lr2.T, preferred_element_type=jnp.float32)


def _c_body(a00_ref, a01_ref, a10_ref, a11_ref,
            hall_ref, hlz_ref, hlr_ref, dinv_ref, h_ref,
            lz_ref, lr_ref, lh_ref, bz_ref, br_ref, bh_ref,
            lzb_ref, lrb_ref, lhb_ref, linw_ref, linb_ref, y_ref, hn_ref):
    dinv = dinv_ref[...]
    hall = hall_ref[...]
    acc = jnp.concatenate([a00_ref[...] + a10_ref[...],
                           a01_ref[...] + a11_ref[...]], axis=1)
    agg = dinv * acc + dinv * dinv * hall
    lz1 = lz_ref[:, :F_H]
    lr1 = lr_ref[:, :F_H]
    lh1 = lh_ref[:, :F_H]
    lh2 = lh_ref[:, F_H:]
    bzf = jnp.dot(bz_ref[...], lz1.T, preferred_element_type=jnp.float32) + lzb_ref[...]
    brf = jnp.dot(br_ref[...], lr1.T, preferred_element_type=jnp.float32) + lrb_ref[...]
    bhf = jnp.dot(bh_ref[...], lh1.T, preferred_element_type=jnp.float32) + lhb_ref[...]
    h = h_ref[...]
    z = jax.nn.sigmoid(agg[:, :F_H] + hlz_ref[...] + bzf)
    r = jax.nn.sigmoid(agg[:, F_H:2 * F_H] + hlr_ref[...] + brf)
    ht = jnp.tanh(agg[:, 2 * F_H:] + jnp.dot(h * r, lh2.T, preferred_element_type=jnp.float32) + bhf)
    hn = z * h + (1.0 - z) * ht
    hn_ref[...] = hn
    y_ref[...] = (jnp.dot(jnp.maximum(hn, 0.0), linw_ref[...].T,
                          preferred_element_type=jnp.float32) + linb_ref[...])


def kernel(x, edge_index, edge_weight, prev_hidden_state,
           Wz, bz, Lz_W, Lz_b, Wr, br, Lr_W, Lr_b,
           Wh, bh, Lh_W, Lh_b, lin_W, lin_b):
    src = edge_index[0]
    dst = edge_index[1]
    pad = EPAD - E
    src2 = jnp.concatenate([src, jnp.zeros((pad,), src.dtype)]).reshape(ROWS_ALL, CHUNK)
    dst2 = jnp.concatenate([dst, jnp.zeros((pad,), dst.dtype)]).reshape(ROWS_ALL, CHUNK)
    w2 = jnp.concatenate([edge_weight, jnp.zeros((pad,), edge_weight.dtype)]).reshape(ROWS_ALL, CHUNK)

    deg_p = _b0_kernel(dst2, w2)

    bn = 1000
    grid = (N // bn,)
    full = lambda shp: pl.BlockSpec(shp, lambda i: (0, 0))
    dg0 = deg_p[0, :N].reshape(N, 1)
    dg1 = deg_p[1, :N].reshape(N, 1)
    hall, hd0, hd1, hlz, hlr, dinv1 = pl.pallas_call(
        _a_body,
        grid=grid,
        in_specs=[
            pl.BlockSpec((bn, F_IN), lambda i: (i, 0)),
            pl.BlockSpec((bn, F_H), lambda i: (i, 0)),
            pl.BlockSpec((bn, 1), lambda i: (i, 0)),
            pl.BlockSpec((bn, 1), lambda i: (i, 0)),
            full((F_H, F_IN)), full((F_H, F_IN)), full((F_H, F_IN)),
            full((F_H, 2 * F_H)), full((F_H, 2 * F_H)), full((F_H, 2 * F_H)),
        ],
        out_specs=[
            pl.BlockSpec((bn, F96), lambda i: (i, 0)),
            pl.BlockSpec((bn, FH2), lambda i: (i, 0)),
            pl.BlockSpec((bn, FH2), lambda i: (i, 0)),
            pl.BlockSpec((bn, F_H), lambda i: (i, 0)),
            pl.BlockSpec((bn, F_H), lambda i: (i, 0)),
            pl.BlockSpec((bn, 1), lambda i: (i, 0)),
        ],
        out_shape=[
            jax.ShapeDtypeStruct((N, F96), jnp.float32),
            jax.ShapeDtypeStruct((N, FH2), jnp.float32),
            jax.ShapeDtypeStruct((N, FH2), jnp.float32),
            jax.ShapeDtypeStruct((N, F_H), jnp.float32),
            jax.ShapeDtypeStruct((N, F_H), jnp.float32),
            jax.ShapeDtypeStruct((N, 1), jnp.float32),
        ],
    )(x, prev_hidden_state, dg0, dg1, Wz, Wr, Wh, Lz_W, Lr_W, Lh_W)

    accp = _b1_kernel(src2, dst2, w2, hd0, hd1)

    a00 = accp[0, 0]
    a01 = accp[0, 1]
    a10 = accp[1, 0]
    a11 = accp[1, 1]
    bz1 = bz.reshape(1, F_H)
    br1 = br.reshape(1, F_H)
    bh1 = bh.reshape(1, F_H)
    lzb1 = Lz_b.reshape(1, F_H)
    lrb1 = Lr_b.reshape(1, F_H)
    lhb1 = Lh_b.reshape(1, F_H)
    linb1 = lin_b.reshape(1, F_OUT)

    y, hn = pl.pallas_call(
        _c_body,
        grid=grid,
        in_specs=[
            pl.BlockSpec((bn, FH2), lambda i: (i, 0)),
            pl.BlockSpec((bn, FH2), lambda i: (i, 0)),
            pl.BlockSpec((bn, FH2), lambda i: (i, 0)),
            pl.BlockSpec((bn, FH2), lambda i: (i, 0)),
            pl.BlockSpec((bn, F96), lambda i: (i, 0)),
            pl.BlockSpec((bn, F_H), lambda i: (i, 0)),
            pl.BlockSpec((bn, F_H), lambda i: (i, 0)),
            pl.BlockSpec((bn, 1), lambda i: (i, 0)),
            pl.BlockSpec((bn, F_H), lambda i: (i, 0)),
            full((F_H, 2 * F_H)),
            full((F_H, 2 * F_H)),
            full((F_H, 2 * F_H)),
            full((1, F_H)), full((1, F_H)), full((1, F_H)),
            full((1, F_H)), full((1, F_H)), full((1, F_H)),
            full((F_OUT, F_H)),
            full((1, F_OUT)),
        ],
        out_specs=[
            pl.BlockSpec((bn, F_OUT), lambda i: (i, 0)),
            pl.BlockSpec((bn, F_H), lambda i: (i, 0)),
        ],
        out_shape=[
            jax.ShapeDtypeStruct((N, F_OUT), jnp.float32),
            jax.ShapeDtypeStruct((N, F_H), jnp.float32),
        ],
    )(a00, a01, a10, a11, hall, hlz, hlr, dinv1, prev_hidden_state,
      Lz_W, Lr_W, Lh_W, bz1, br1, bh1, lzb1, lrb1, lhb1,
      lin_W, linb1)

    return (y, hn)


# speedup vs baseline: 1.0414x; 1.0215x over previous
import functools

import jax
import jax.numpy as jnp
from jax import lax
from jax.experimental import pallas as pl
from jax.experimental.pallas import tpu as pltpu
from jax.experimental.pallas import tpu_sc as plsc

N = 10000
NP = 10240
E = 320000
F_IN = 128
F96 = 96
F_H = 32
F_OUT = 45

CHUNK = 128
TILES = 32
G_B1 = 80
EPAD = TILES * G_B1 * CHUNK
ROWS_ALL = EPAD // CHUNK
G_B0 = ROWS_ALL // 16
NSLICE = NP // 16


def _zeros16():
    return jnp.zeros((16,), jnp.float32)


def _b0_body(dst_ref, w_ref, deg_ref, deg_acc, dv, wv, zbuf):
    c = lax.axis_index("c")
    s = lax.axis_index("s")
    wid = s * 2 + c

    def zb(i, _):
        zbuf[pl.ds(i * 16, 16)] = _zeros16()
        return _
    lax.fori_loop(0, NSLICE // 16, zb, None)
    pltpu.sync_copy(zbuf, deg_acc.at[pl.ds(s * NSLICE, NSLICE)])
    pltpu.sync_copy(dst_ref.at[pl.ds(wid * G_B1, G_B1)], dv)
    pltpu.sync_copy(w_ref.at[pl.ds(wid * G_B1, G_B1)], wv)
    plsc.subcore_barrier()

    def scat(j, _):
        pltpu.sync_copy(wv.at[j], deg_acc.at[dv.at[j]], add=True)
        return _
    lax.fori_loop(0, G_B1, scat, None)
    plsc.subcore_barrier()
    pltpu.sync_copy(deg_acc.at[pl.ds(s * NSLICE, NSLICE)],
                    deg_ref.at[c, pl.ds(s * NSLICE, NSLICE)])


_SC_PARAMS = pltpu.CompilerParams(needs_layout_passes=False,
                                  use_tc_tiling_on_sc=False)

_b0_kernel = functools.partial(
    pl.kernel,
    mesh=plsc.VectorSubcoreMesh(core_axis_name="c", subcore_axis_name="s"),
    compiler_params=_SC_PARAMS,
    out_type=jax.ShapeDtypeStruct((2, NP), jnp.float32),
    scratch_types=[
        pltpu.VMEM_SHARED((NP,), jnp.float32),
        pltpu.VMEM((G_B1, CHUNK), jnp.int32),
        pltpu.VMEM((G_B1, CHUNK), jnp.float32),
        pltpu.VMEM((NSLICE,), jnp.float32),
    ],
)(_b0_body)


FH2 = F96 // 2


def _b1_body(src_ref, dst_ref, w_ref, hd0_ref, hd1_ref, acc_ref,
             hd_half, acc_half, srcv, dstv, wv, rows0, rows1, zbuf,
             gsem0, gsem1, ssem0, ssem1):
    c = lax.axis_index("c")
    s = lax.axis_index("s")
    wid = s * 2 + c
    row0 = wid * G_B1

    pltpu.sync_copy(src_ref.at[pl.ds(row0, G_B1)], srcv)
    pltpu.sync_copy(dst_ref.at[pl.ds(row0, G_B1)], dstv)
    pltpu.sync_copy(w_ref.at[pl.ds(row0, G_B1)], wv)

    def zb(i, _):
        for j in range(FH2 // 16):
            zbuf[i, pl.ds(j * 16, 16)] = _zeros16()
        return _
    lax.fori_loop(0, 64, zb, None)

    def scale_buf(g, rows):
        def sc(e8, _):
            for de in range(8):
                e = e8 * 8 + de
                sp = plsc.load_gather(
                    wv, [jnp.full((16,), g, jnp.int32),
                         jnp.full((16,), e, jnp.int32)])
                for j in range(FH2 // 16):
                    rows[e, pl.ds(j * 16, 16)] = rows[e, pl.ds(j * 16, 16)] * sp
            return _
        lax.fori_loop(0, CHUNK // 8, sc, None)

    for h, hd_h in ((0, hd0_ref), (1, hd1_ref)):
        def zc(i, _):
            pltpu.sync_copy(zbuf, acc_half.at[pl.ds(s * NSLICE + i * 64, 64)])
            return _
        lax.fori_loop(0, NSLICE // 64, zc, None)

        @pl.when(s < 15)
        def _():
            pltpu.sync_copy(hd_h.at[pl.ds(s * NSLICE, NSLICE)],
                            hd_half.at[pl.ds(s * NSLICE, NSLICE)])

        @pl.when(s == 15)
        def _():
            pltpu.sync_copy(hd_h.at[pl.ds(15 * NSLICE, N - 15 * NSLICE)],
                            hd_half.at[pl.ds(15 * NSLICE, N - 15 * NSLICE)])

        plsc.subcore_barrier()

        pltpu.async_copy(hd_half.at[srcv.at[0]], rows0, gsem0)
        pltpu.async_copy(hd_half.at[srcv.at[1]], rows1, gsem1)

        def pipe(gp, _):
            g0 = gp * 2
            g1 = g0 + 1
            pltpu.make_async_copy(hd_half.at[srcv.at[g0]], rows0, gsem0).wait()
            scale_buf(g0, rows0)
            pltpu.make_async_copy(hd_half.at[srcv.at[g1]], rows1, gsem1).wait()
            pltpu.async_copy(rows0, acc_half.at[dstv.at[g0]], ssem0, add=True)
            scale_buf(g1, rows1)
            pltpu.async_copy(rows1, acc_half.at[dstv.at[g1]], ssem1, add=True)

            @pl.when(g0 + 2 < G_B1)
            def _():
                pltpu.make_async_copy(rows0, acc_half.at[dstv.at[g0]], ssem0).wait()
                pltpu.async_copy(hd_half.at[srcv.at[g0 + 2]], rows0, gsem0)
                pltpu.make_async_copy(rows1, acc_half.at[dstv.at[g1]], ssem1).wait()
                pltpu.async_copy(hd_half.at[srcv.at[g1 + 2]], rows1, gsem1)
            return _
        lax.fori_loop(0, G_B1 // 2, pipe, None)
        pltpu.make_async_copy(rows0, acc_half.at[dstv.at[G_B1 - 2]], ssem0).wait()
        pltpu.make_async_copy(rows1, acc_half.at[dstv.at[G_B1 - 1]], ssem1).wait()
        plsc.subcore_barrier()

        pltpu.sync_copy(acc_half.at[pl.ds(s * NSLICE, NSLICE)],
                        acc_ref.at[c, h, pl.ds(s * NSLICE, NSLICE)])


_b1_kernel = functools.partial(
    pl.kernel,
    mesh=plsc.VectorSubcoreMesh(core_axis_name="c", subcore_axis_name="s"),
    compiler_params=_SC_PARAMS,
    out_type=jax.ShapeDtypeStruct((2, 2, NP, FH2), jnp.float32),
    scratch_types=[
        pltpu.VMEM_SHARED((NP, FH2), jnp.float32),
        pltpu.VMEM_SHARED((NP, FH2), jnp.float32),
        pltpu.VMEM((G_B1, CHUNK), jnp.int32),
        pltpu.VMEM((G_B1, CHUNK), jnp.int32),
        pltpu.VMEM((G_B1, CHUNK), jnp.float32),
        pltpu.VMEM((CHUNK, FH2), jnp.float32),
        pltpu.VMEM((CHUNK, FH2), jnp.float32),
        pltpu.VMEM((64, FH2), jnp.float32),
        pltpu.SemaphoreType.DMA,
        pltpu.SemaphoreType.DMA,
        pltpu.SemaphoreType.DMA,
        pltpu.SemaphoreType.DMA,
    ],
)(_b1_body)


def _a_body(x_ref, dg0_ref, dg1_ref, wz_ref, wr_ref, wh_ref,
            lz_ref, lr_ref, lh_ref,
            hd0_ref, hd1_ref, dinv_ref):
    lz1 = lz_ref[:, :F_H]
    lr1 = lr_ref[:, :F_H]
    lh1 = lh_ref[:, :F_H]
    wf = jnp.concatenate([
        jnp.dot(lz1, wz_ref[...], preferred_element_type=jnp.float32),
        jnp.dot(lr1, wr_ref[...], preferred_element_type=jnp.float32),
        jnp.dot(lh1, wh_ref[...], preferred_element_type=jnp.float32),
    ], axis=0)
    hall = jnp.dot(x_ref[...], wf.T, preferred_element_type=jnp.float32)
    dinv = lax.rsqrt(dg0_ref[...] + dg1_ref[...] + 1.0)
    dinv_ref[...] = dinv
    hd = hall * dinv
    hd0_ref[...] = hd[:, :FH2]
    hd1_ref[...] = hd[:, FH2:]


def _c_body(a00_ref, a01_ref, a10_ref, a11_ref,
            hd0_ref, hd1_ref, dinv_ref, h_ref,
            lz_ref, lr_ref, lh_ref, bz_ref, br_ref, bh_ref,
            lzb_ref, lrb_ref, lhb_ref, linw_ref, linb_ref, y_ref, hn_ref):
    dinv = dinv_ref[...]
    agg = dinv * jnp.concatenate(
        [a00_ref[...] + a10_ref[...] + hd0_ref[...],
         a01_ref[...] + a11_ref[...] + hd1_ref[...]], axis=1)
    lz1 = lz_ref[:, :F_H]
    lz2 = lz_ref[:, F_H:]
    lr1 = lr_ref[:, :F_H]
    lr2 = lr_ref[:, F_H:]
    lh1 = lh_ref[:, :F_H]
    lh2 = lh_ref[:, F_H:]
    bzf = jnp.dot(bz_ref[...], lz1.T, preferred_element_type=jnp.float32) + lzb_ref[...]
    brf = jnp.dot(br_ref[...], lr1.T, preferred_element_type=jnp.float32) + lrb_ref[...]
    bhf = jnp.dot(bh_ref[...], lh1.T, preferred_element_type=jnp.float32) + lhb_ref[...]
    h = h_ref[...]
    hlz = jnp.dot(h, lz2.T, preferred_element_type=jnp.float32)
    hlr = jnp.dot(h, lr2.T, preferred_element_type=jnp.float32)
    z = jax.nn.sigmoid(agg[:, :F_H] + hlz + bzf)
    r = jax.nn.sigmoid(agg[:, F_H:2 * F_H] + hlr + brf)
    ht = jnp.tanh(agg[:, 2 * F_H:] + jnp.dot(h * r, lh2.T, preferred_element_type=jnp.float32) + bhf)
    hn = z * h + (1.0 - z) * ht
    hn_ref[...] = hn
    y_ref[...] = (jnp.dot(jnp.maximum(hn, 0.0), linw_ref[...].T,
                          preferred_element_type=jnp.float32) + linb_ref[...])


def kernel(x, edge_index, edge_weight, prev_hidden_state,
           Wz, bz, Lz_W, Lz_b, Wr, br, Lr_W, Lr_b,
           Wh, bh, Lh_W, Lh_b, lin_W, lin_b):
    src = edge_index[0]
    dst = edge_index[1]
    pad = EPAD - E
    src2 = jnp.concatenate([src, jnp.zeros((pad,), src.dtype)]).reshape(ROWS_ALL, CHUNK)
    dst2 = jnp.concatenate([dst, jnp.zeros((pad,), dst.dtype)]).reshape(ROWS_ALL, CHUNK)
    w2 = jnp.concatenate([edge_weight, jnp.zeros((pad,), edge_weight.dtype)]).reshape(ROWS_ALL, CHUNK)

    deg_p = _b0_kernel(dst2, w2)

    bn = 1000
    grid = (N // bn,)
    full = lambda shp: pl.BlockSpec(shp, lambda i: (0, 0))
    dg0 = deg_p[0, :N].reshape(N, 1)
    dg1 = deg_p[1, :N].reshape(N, 1)
    hd0, hd1, dinv1 = pl.pallas_call(
        _a_body,
        grid=grid,
        in_specs=[
            pl.BlockSpec((bn, F_IN), lambda i: (i, 0)),
            pl.BlockSpec((bn, 1), lambda i: (i, 0)),
            pl.BlockSpec((bn, 1), lambda i: (i, 0)),
            full((F_H, F_IN)), full((F_H, F_IN)), full((F_H, F_IN)),
            full((F_H, 2 * F_H)), full((F_H, 2 * F_H)), full((F_H, 2 * F_H)),
        ],
        out_specs=[
            pl.BlockSpec((bn, FH2), lambda i: (i, 0)),
            pl.BlockSpec((bn, FH2), lambda i: (i, 0)),
            pl.BlockSpec((bn, 1), lambda i: (i, 0)),
        ],
        out_shape=[
            jax.ShapeDtypeStruct((N, FH2), jnp.float32),
            jax.ShapeDtypeStruct((N, FH2), jnp.float32),
            jax.ShapeDtypeStruct((N, 1), jnp.float32),
        ],
    )(x, dg0, dg1, Wz, Wr, Wh, Lz_W, Lr_W, Lh_W)

    accp = _b1_kernel(src2, dst2, w2, hd0, hd1)

    a00 = accp[0, 0]
    a01 = accp[0, 1]
    a10 = accp[1, 0]
    a11 = accp[1, 1]
    bz1 = bz.reshape(1, F_H)
    br1 = br.reshape(1, F_H)
    bh1 = bh.reshape(1, F_H)
    lzb1 = Lz_b.reshape(1, F_H)
    lrb1 = Lr_b.reshape(1, F_H)
    lhb1 = Lh_b.reshape(1, F_H)
    linb1 = lin_b.reshape(1, F_OUT)

    y, hn = pl.pallas_call(
        _c_body,
        grid=grid,
        in_specs=[
            pl.BlockSpec((bn, FH2), lambda i: (i, 0)),
            pl.BlockSpec((bn, FH2), lambda i: (i, 0)),
            pl.BlockSpec((bn, FH2), lambda i: (i, 0)),
            pl.BlockSpec((bn, FH2), lambda i: (i, 0)),
            pl.BlockSpec((bn, FH2), lambda i: (i, 0)),
            pl.BlockSpec((bn, FH2), lambda i: (i, 0)),
            pl.BlockSpec((bn, 1), lambda i: (i, 0)),
            pl.BlockSpec((bn, F_H), lambda i: (i, 0)),
            full((F_H, 2 * F_H)),
            full((F_H, 2 * F_H)),
            full((F_H, 2 * F_H)),
            full((1, F_H)), full((1, F_H)), full((1, F_H)),
            full((1, F_H)), full((1, F_H)), full((1, F_H)),
            full((F_OUT, F_H)),
            full((1, F_OUT)),
        ],
        out_specs=[
            pl.BlockSpec((bn, F_OUT), lambda i: (i, 0)),
            pl.BlockSpec((bn, F_H), lambda i: (i, 0)),
        ],
        out_shape=[
            jax.ShapeDtypeStruct((N, F_OUT), jnp.float32),
            jax.ShapeDtypeStruct((N, F_H), jnp.float32),
        ],
    )(a00, a01, a10, a11, hd0, hd1, dinv1, prev_hidden_state,
      Lz_W, Lr_W, Lh_W, bz1, br1, bh1, lzb1, lrb1, lhb1,
      lin_W, linb1)

    return (y, hn)
